# baseline (speedup 1.0000x reference)
import jax
import jax.numpy as jnp
from jax import lax
from jax.experimental import pallas as pl
from jax.experimental.pallas import tpu as pltpu
from jax.experimental.pallas import tpu_sc as plsc

N_ITEM = 40000
N_USER = 10000
N = N_ITEM + N_USER
D = 128
NNZ = 200000

NC = 2
NS = 16
L = 16

E_TOT = 3 * NNZ
E_PAD = 614400
CHUNK = E_PAD // NS
EB = 128
MAC = 3 * EB

P = 5
R = 5120
ACC_ROWS = R + L
MSG_PAD = NC * P * R

SUB = 2400
NSUB = CHUNK // SUB
CAP = 7680
THRESH = CAP - SUB
CSIZE = CAP + 400


def _mm3_body(h_ref, w_ref, o_ref):
    o_ref[...] = lax.dot_general(
        h_ref[...], w_ref[0],
        (((1,), (1,)), ((), ())),
        preferred_element_type=jnp.float32)


def _transform_all(h_all, w_stack):
    blk = 1000
    nb = N // blk
    return pl.pallas_call(
        _mm3_body,
        grid=(3, nb),
        in_specs=[
            pl.BlockSpec((blk, D), lambda r, i: (i, 0)),
            pl.BlockSpec((1, D, D), lambda r, i: (r, 0, 0)),
        ],
        out_specs=pl.BlockSpec((blk, D), lambda r, i: (r * nb + i, 0)),
        out_shape=jax.ShapeDtypeStruct((3 * N, D), jnp.float32),
    )(h_all, w_stack)


def _sc_body(x_hbm, src_hbm, dst_hbm, val_hbm, msg_hbm,
             acc, r0, r1, r2, csrc, cdl, cval,
             ea0, ea1, ea2, eb0, eb1, eb2, dloc,
             g0, g1, g2, esa, esb):
    c = lax.axis_index("c")
    s = lax.axis_index("s")
    iota = lax.iota(jnp.int32, L)
    rbufs = (r0, r1, r2)
    gsems = (g0, g1, g2)
    ebufs_a = (ea0, ea1, ea2)
    ebufs_b = (eb0, eb1, eb2)
    lane_ids = [jnp.full((L, 1), l, dtype=jnp.int32) for l in range(L)]
    gdn = lax.GatherDimensionNumbers(
        offset_dims=(), collapsed_slice_dims=(0,), start_index_map=(0,))

    def _lane_bcast(v, l):
        return lax.gather(v, lane_ids[l], gdn, slice_sizes=(1,),
                          mode=lax.GatherScatterMode.PROMISE_IN_BOUNDS)

    def _g_start(b, u):
        pltpu.async_copy(x_hbm.at[csrc.at[pl.ds(b * EB, EB)]],
                         rbufs[u], gsems[u])

    def _g_wait(b, u):
        pltpu.make_async_copy(x_hbm.at[csrc.at[pl.ds(b * EB, EB)]],
                              rbufs[u], gsems[u]).wait()

    def _scale(b, u):
        rbuf = rbufs[u]

        def _group(i, carry):
            v = cval[pl.ds(b * EB + i * L, L)]
            for l in range(L):
                bc = _lane_bcast(v, l)
                r = i * L + l
                for q in range(8):
                    rbuf[r, pl.ds(q * L, L)] = rbuf[r, pl.ds(q * L, L)] * bc
            return carry
        lax.fori_loop(0, EB // L, _group, 0)

    def _drain_pipe(nb3):
        _g_start(0, 0)
        _g_start(1, 1)
        _g_start(2, 2)

        def _macro(m, carry):
            for u in range(3):
                b = 3 * m + u
                _g_wait(b, u)
                _scale(b, u)
                for i in range(EB // L):
                    dloc[u, pl.ds(i * L, L)] = cdl[pl.ds(b * EB + i * L, L)]
                pltpu.sync_copy(rbufs[u], acc.at[dloc.at[u]], add=True)
                bn = jnp.minimum(b + 3, 3 * nb3 - 1)
                _g_start(bn, u)
            return carry
        lax.fori_loop(0, nb3, _macro, 0)
        for u in range(3):
            _g_wait(0, u)

    def _mid_drain(cnt):
        nb3 = cnt // MAC
        _drain_pipe(nb3)
        base = nb3 * MAC
        for j in range(MAC // L):
            csrc[pl.ds(j * L, L)] = csrc[pl.ds(base + j * L, L)]
            cdl[pl.ds(j * L, L)] = cdl[pl.ds(base + j * L, L)]
            cval[pl.ds(j * L, L)] = cval[pl.ds(base + j * L, L)]
        return cnt - base

    trash = jnp.full((L,), CSIZE - L, dtype=jnp.int32) + iota
    shifts = []
    for k in (1, 2, 4, 8):
        shifts.append((jnp.maximum(iota - k, 0).reshape(L, 1), iota >= k))

    def _compress(bufs, lo, cnt):
        e_src, e_dst, e_val = bufs

        def _grp(i, cnt2):
            v_dst = e_dst[pl.ds(i * L, L)]
            v_src = e_src[pl.ds(i * L, L)]
            v_val = e_val[pl.ds(i * L, L)]
            m = (v_dst >= lo) & (v_dst < lo + R)
            x = jnp.where(m, 1, 0)
            for idx, keep in shifts:
                sh = lax.gather(x, idx, gdn, slice_sizes=(1,),
                                mode=lax.GatherScatterMode.PROMISE_IN_BOUNDS)
                x = x + jnp.where(keep, sh, 0)
            pos = jnp.where(m, cnt2 + x - 1, trash)
            plsc.store_scatter(csrc, [pos], v_src)
            plsc.store_scatter(cval, [pos], v_val)
            plsc.store_scatter(cdl, [pos], v_dst - lo)
            return cnt2 + plsc.all_reduce_population_count(m)[0]
        return lax.fori_loop(0, SUB // L, _grp, cnt)

    def _e_start(k, bufs, sem):
        e0 = s * CHUNK + k * SUB
        pltpu.async_copy(src_hbm.at[pl.ds(e0, SUB)], bufs[0], sem)
        pltpu.async_copy(dst_hbm.at[pl.ds(e0, SUB)], bufs[1], sem)
        pltpu.async_copy(val_hbm.at[pl.ds(e0, SUB)], bufs[2], sem)

    def _e_wait(k, bufs, sem):
        e0 = s * CHUNK + k * SUB
        pltpu.make_async_copy(src_hbm.at[pl.ds(e0, SUB)], bufs[0], sem).wait()
        pltpu.make_async_copy(dst_hbm.at[pl.ds(e0, SUB)], bufs[1], sem).wait()
        pltpu.make_async_copy(val_hbm.at[pl.ds(e0, SUB)], bufs[2], sem).wait()

    def _pass(p, carry):
        lo = (c * P + p) * R

        def _zrow(i, cy):
            for q in range(8):
                r0[i, pl.ds(q * L, L)] = jnp.zeros((L,), jnp.float32)
            return cy
        lax.fori_loop(0, EB, _zrow, 0)
        tile_rows = ACC_ROWS // NS
        zb = s * tile_rows
        for z in range(tile_rows // EB):
            pltpu.sync_copy(r0, acc.at[pl.ds(zb + z * EB, EB)])
        zrem = tile_rows % EB
        if zrem:
            pltpu.sync_copy(r0.at[pl.ds(0, zrem)],
                            acc.at[pl.ds(zb + tile_rows - zrem, zrem)])
        plsc.subcore_barrier()

        _e_start(0, ebufs_a, esa)

        def _scan(m, cnt):
            ka = 2 * m
            _e_wait(ka, ebufs_a, esa)
            _e_start(2 * m + 1, ebufs_b, esb)
            cnt = _compress(ebufs_a, lo, cnt)
            cnt = lax.cond(cnt > THRESH, _mid_drain, lambda x: x, cnt)
            _e_wait(2 * m + 1, ebufs_b, esb)
            ka2 = jnp.minimum(2 * m + 2, NSUB - 1)
            _e_start(ka2, ebufs_a, esa)
            cnt = _compress(ebufs_b, lo, cnt)
            cnt = lax.cond(cnt > THRESH, _mid_drain, lambda x: x, cnt)
            return cnt
        cnt = lax.fori_loop(0, NSUB // 2, _scan, jnp.int32(0))
        _e_wait(NSUB - 1, ebufs_a, esa)

        for j in range(MAC // L):
            pos = cnt + j * L
            csrc[pl.ds(pos, L)] = jnp.zeros((L,), jnp.int32)
            cval[pl.ds(pos, L)] = jnp.zeros((L,), jnp.float32)
            cdl[pl.ds(pos, L)] = R + iota
        nb3 = jnp.maximum((cnt + MAC - 1) // MAC, 1)
        _drain_pipe(nb3)
        plsc.subcore_barrier()

        rpt = R // NS
        pltpu.sync_copy(acc.at[pl.ds(s * rpt, rpt)],
                        msg_hbm.at[pl.ds(lo + s * rpt, rpt)])
        plsc.subcore_barrier()
        return carry

    lax.fori_loop(0, P, _pass, 0)


def _message_pass(x, src, dst, val):
    mesh = plsc.VectorSubcoreMesh(core_axis_name="c", subcore_axis_name="s",
                                  num_cores=NC, num_subcores=NS)
    f = pl.kernel(
        _sc_body,
        out_type=jax.ShapeDtypeStruct((MSG_PAD, D), jnp.float32),
        mesh=mesh,
        compiler_params=pltpu.CompilerParams(needs_layout_passes=False),
        scratch_types=[
            pltpu.VMEM_SHARED((ACC_ROWS, D), jnp.float32),
            pltpu.VMEM((EB, D), jnp.float32),
            pltpu.VMEM((EB, D), jnp.float32),
            pltpu.VMEM((EB, D), jnp.float32),
            pltpu.VMEM((CSIZE,), jnp.int32),
            pltpu.VMEM((CSIZE,), jnp.int32),
            pltpu.VMEM((CSIZE,), jnp.float32),
            pltpu.VMEM((SUB,), jnp.int32),
            pltpu.VMEM((SUB,), jnp.int32),
            pltpu.VMEM((SUB,), jnp.float32),
            pltpu.VMEM((SUB,), jnp.int32),
            pltpu.VMEM((SUB,), jnp.int32),
            pltpu.VMEM((SUB,), jnp.float32),
            pltpu.VMEM((3, EB), jnp.int32),
            pltpu.SemaphoreType.DMA,
            pltpu.SemaphoreType.DMA,
            pltpu.SemaphoreType.DMA,
            pltpu.SemaphoreType.DMA,
            pltpu.SemaphoreType.DMA,
        ],
    )
    return f(x, src, dst, val)


def _out_body(m_ref, h_ref, wi_ref, wu_ref, bi_ref, bu_ref, o_ref):
    i = pl.program_id(0)
    z = m_ref[...] + h_ref[...]

    def _apply(w, b):
        o_ref[...] = jnp.maximum(
            lax.dot_general(z, w, (((1,), (1,)), ((), ())),
                            preferred_element_type=jnp.float32) + b, 0.0)

    pl.when(i < N_ITEM // 400)(lambda: _apply(wi_ref[...], bi_ref[...]))
    pl.when(i >= N_ITEM // 400)(lambda: _apply(wu_ref[...], bu_ref[...]))


def _output_layer(msg, h_all, w_item, b_item, w_user, b_user):
    blk = 400
    return pl.pallas_call(
        _out_body,
        grid=(N // blk,),
        in_specs=[
            pl.BlockSpec((blk, D), lambda i: (i, 0)),
            pl.BlockSpec((blk, D), lambda i: (i, 0)),
            pl.BlockSpec((D, D), lambda i: (0, 0)),
            pl.BlockSpec((D, D), lambda i: (0, 0)),
            pl.BlockSpec((1, D), lambda i: (0, 0)),
            pl.BlockSpec((1, D), lambda i: (0, 0)),
        ],
        out_specs=pl.BlockSpec((blk, D), lambda i: (i, 0)),
        out_shape=jax.ShapeDtypeStruct((N, D), jnp.float32),
    )(msg, h_all, w_item, w_user, b_item.reshape(1, D), b_user.reshape(1, D))


def kernel(h_item, h_user, A0_values, A1_values, A2_values,
           W_r0, W_r1, W_r2, W_item, b_item, W_user, b_user,
           A0_indices, A1_indices, A2_indices):
    h_all = jnp.concatenate([h_item, h_user], axis=0)
    w_stack = jnp.stack([W_r0, W_r1, W_r2], axis=0)

    x = _transform_all(h_all, w_stack)

    src = jnp.concatenate([
        A0_indices[1].astype(jnp.int32),
        A1_indices[1].astype(jnp.int32) + N,
        A2_indices[1].astype(jnp.int32) + 2 * N,
    ])
    dst = jnp.concatenate([
        A0_indices[0].astype(jnp.int32),
        A1_indices[0].astype(jnp.int32),
        A2_indices[0].astype(jnp.int32),
    ])
    val = jnp.concatenate([A0_values, A1_values, A2_values])
    pad = E_PAD - E_TOT
    src = jnp.concatenate([src, jnp.zeros((pad,), jnp.int32)])
    dst = jnp.concatenate([dst, jnp.zeros((pad,), jnp.int32)])
    val = jnp.concatenate([val, jnp.zeros((pad,), jnp.float32)])

    msg = _message_pass(x, src, dst, val)[:N]

    out = _output_layer(msg, h_all, W_item, b_item, W_user, b_user)
    return (out[:N_ITEM], out[N_ITEM:])
